# baseline (speedup 1.0000x reference)
import jax
import jax.numpy as jnp
from jax import lax
from jax.experimental import pallas as pl
from jax.experimental.pallas import tpu as pltpu
from jax.experimental.pallas import tpu_sc as plsc

_VOCAB = 10000
_ACTIVE = 100
_DIM = 10
_L = 16
_NW = 32
_BATCH = 16384
_HIST = 200
_ROWS_W = _BATCH // _NW
_CB = 16
_NCHUNK = _ROWS_W // _CB


def _sc_body(idx_hbm, lookup_hbm, emb_hbm, out_hbm,
             lookup_v, emb_v, idx_v, out_v):
    wid = lax.axis_index("s") * 2 + lax.axis_index("c")
    row0 = wid * _ROWS_W

    pltpu.sync_copy(lookup_hbm, lookup_v)
    pltpu.sync_copy(emb_hbm, emb_v)

    lanes = lax.iota(jnp.int32, _L)

    def chunk_body(c, carry):
        b0 = row0 + c * _CB
        pltpu.sync_copy(idx_hbm.at[pl.ds(b0, _CB)], idx_v)

        @plsc.parallel_loop(0, _HIST, 1, unroll=4)
        def h_body(h):
            hvec = jnp.broadcast_to(h, (_L,))
            ids = plsc.load_gather(idx_v, [lanes, hvec])
            rem = plsc.load_gather(lookup_v, [ids])
            rem = jnp.minimum(jnp.maximum(rem, 0), _ACTIVE - 1)
            for j in range(_DIM):
                colj = jnp.full((_L,), j, jnp.int32)
                vals = plsc.load_gather(emb_v, [rem, colj])
                plsc.store_scatter(out_v, [lanes, hvec, colj], vals)

        pltpu.sync_copy(out_v, out_hbm.at[pl.ds(b0, _CB)])
        return carry

    lax.fori_loop(0, _NCHUNK, chunk_body, 0, unroll=False)


def kernel(input_indices, lookup, emb_weight):
    mesh = plsc.VectorSubcoreMesh(core_axis_name="c", subcore_axis_name="s")
    f = pl.kernel(
        _sc_body,
        out_type=jax.ShapeDtypeStruct((_BATCH, _HIST, _DIM), jnp.float32),
        mesh=mesh,
        compiler_params=pltpu.CompilerParams(needs_layout_passes=False,
                                             use_tc_tiling_on_sc=False),
        scratch_types=[
            pltpu.VMEM((_VOCAB,), jnp.int32),
            pltpu.VMEM((_ACTIVE, _DIM), jnp.float32),
            pltpu.VMEM((_CB, _HIST), jnp.int32),
            pltpu.VMEM((_CB, _HIST, _DIM), jnp.float32),
        ],
    )
    return f(input_indices, lookup, emb_weight)


# speedup vs baseline: 13.0999x; 13.0999x over previous
import jax
import jax.numpy as jnp
from jax import lax
from jax.experimental import pallas as pl
from jax.experimental.pallas import tpu as pltpu
from jax.experimental.pallas import tpu_sc as plsc

_VOCAB = 10000
_ACTIVE = 100
_DIM = 10
_L = 16
_NW = 32
_BATCH = 16384
_HIST = 200
_TH = _HIST // 8
_TW = _BATCH // _NW


def _sc_body(idx_hbm, lookup_hbm, emb_hbm, out_hbm,
             lookup_v, emb_v, idx_v, out_v):
    wid = lax.axis_index("s") * 2 + lax.axis_index("c")
    b0 = wid * _TW

    pltpu.sync_copy(lookup_hbm, lookup_v)
    pltpu.sync_copy(emb_hbm, emb_v)

    def th_body(th, carry):
        h0 = th * 8
        pltpu.sync_copy(idx_hbm.at[pl.ds(h0, 8), pl.ds(b0, _TW)], idx_v)

        for r in range(8):
            @plsc.parallel_loop(0, _TW // _L, 1, unroll=4)
            def g_body(g):
                off = pl.multiple_of(g * _L, _L)
                ids = idx_v[r, pl.ds(off, _L)]
                rem = plsc.load_gather(lookup_v, [ids])
                rem = jnp.minimum(jnp.maximum(rem, 0), _ACTIVE - 1)
                rem10 = rem * _DIM
                for d in range(_DIM):
                    vals = plsc.load_gather(emb_v, [rem10 + d])
                    out_v[d, r, pl.ds(off, _L)] = vals

        for d in range(_DIM):
            pltpu.sync_copy(out_v.at[d],
                            out_hbm.at[d, pl.ds(h0, 8), pl.ds(b0, _TW)])
        return carry

    lax.fori_loop(0, _TH, th_body, 0, unroll=False)


def kernel(input_indices, lookup, emb_weight):
    idx_t = input_indices.T
    emb_flat = emb_weight.reshape(-1)
    mesh = plsc.VectorSubcoreMesh(core_axis_name="c", subcore_axis_name="s")
    f = pl.kernel(
        _sc_body,
        out_type=jax.ShapeDtypeStruct((_DIM, _HIST, _BATCH), jnp.float32),
        mesh=mesh,
        compiler_params=pltpu.CompilerParams(needs_layout_passes=False,
                                             use_tc_tiling_on_sc=True),
        scratch_types=[
            pltpu.VMEM((_VOCAB,), jnp.int32),
            pltpu.VMEM((_ACTIVE * _DIM,), jnp.float32),
            pltpu.VMEM((8, _TW), jnp.int32),
            pltpu.VMEM((_DIM, 8, _TW), jnp.float32),
        ],
    )
    out = f(idx_t, lookup, emb_flat)
    return out.transpose(2, 1, 0)


# speedup vs baseline: 14.0937x; 1.0759x over previous
import jax
import jax.numpy as jnp
from jax import lax
from jax.experimental import pallas as pl
from jax.experimental.pallas import tpu as pltpu
from jax.experimental.pallas import tpu_sc as plsc

_VOCAB = 10000
_ACTIVE = 100
_DIM = 10
_L = 16
_NW = 32
_BATCH = 16384
_HIST = 200
_TH = _HIST // 8
_TW = _BATCH // _NW


def _sc_body(idx_hbm, lookup_hbm, emb_hbm, out_hbm,
             lookup_v, emb_v, idx_v, out_v):
    wid = lax.axis_index("s") * 2 + lax.axis_index("c")
    b0 = wid * _TW

    pltpu.sync_copy(lookup_hbm, lookup_v)
    pltpu.sync_copy(emb_hbm, emb_v)

    def th_body(th, carry):
        h0 = th * 8
        pltpu.sync_copy(idx_hbm.at[pl.ds(h0, 8), pl.ds(b0, _TW)], idx_v)

        for r in range(8):
            @plsc.parallel_loop(0, _TW // _L, 1, unroll=4)
            def g_body(g):
                off = pl.multiple_of(g * _L, _L)
                ids = idx_v[r, pl.ds(off, _L)]
                rem = plsc.load_gather(lookup_v, [ids])
                rem = jnp.minimum(jnp.maximum(rem, 0), _ACTIVE - 1)
                rem10 = rem * _DIM
                for d in range(_DIM):
                    vals = plsc.load_gather(emb_v, [rem10 + d])
                    out_v[d, r, pl.ds(off, _L)] = vals

        pltpu.sync_copy(out_v,
                        out_hbm.at[:, pl.ds(h0, 8), pl.ds(b0, _TW)])
        return carry

    lax.fori_loop(0, _TH, th_body, 0, unroll=False)


def kernel(input_indices, lookup, emb_weight):
    idx_t = input_indices.T
    emb_flat = emb_weight.reshape(-1)
    mesh = plsc.VectorSubcoreMesh(core_axis_name="c", subcore_axis_name="s")
    f = pl.kernel(
        _sc_body,
        out_type=jax.ShapeDtypeStruct((_DIM, _HIST, _BATCH), jnp.float32),
        mesh=mesh,
        compiler_params=pltpu.CompilerParams(needs_layout_passes=False,
                                             use_tc_tiling_on_sc=True),
        scratch_types=[
            pltpu.VMEM((_VOCAB,), jnp.int32),
            pltpu.VMEM((_ACTIVE * _DIM,), jnp.float32),
            pltpu.VMEM((8, _TW), jnp.int32),
            pltpu.VMEM((_DIM, 8, _TW), jnp.float32),
        ],
    )
    out = f(idx_t, lookup, emb_flat)
    return out.transpose(2, 1, 0)


# speedup vs baseline: 27.3937x; 1.9437x over previous
import jax
import jax.numpy as jnp
from jax import lax
from jax.experimental import pallas as pl
from jax.experimental.pallas import tpu as pltpu
from jax.experimental.pallas import tpu_sc as plsc

_VOCAB = 10000
_ACTIVE = 100
_DIM = 10
_L = 16
_NW = 32
_BATCH = 16384
_HIST = 200
_TW = _BATCH // _NW
_CW = 256
_NCH = (_HIST // 8) * (_TW // _CW)
_GR = 8 * (_CW // _L)


def _sc_body(idx_hbm, lookup_hbm, emb_hbm, out_hbm,
             lookup_v, emb_v, idx_v, out_v, sin0, sin1, sout0, sout1):
    wid = lax.axis_index("s") * 2 + lax.axis_index("c")
    b0 = wid * _TW
    sin = (sin0, sin1)
    sout = (sout0, sout1)

    pltpu.sync_copy(lookup_hbm, lookup_v)
    pltpu.sync_copy(emb_hbm, emb_v)

    def in_copy(c, s):
        th = c >> 1
        bc = b0 + (c & 1) * _CW
        return pltpu.make_async_copy(
            idx_hbm.at[pl.ds(th * 8, 8), pl.ds(bc, _CW)], idx_v.at[s], sin[s])

    def out_copy(c, s):
        th = c >> 1
        bc = b0 + (c & 1) * _CW
        return pltpu.make_async_copy(
            out_v.at[s], out_hbm.at[:, pl.ds(th * 8, 8), pl.ds(bc, _CW)],
            sout[s])

    in_copy(0, 0).start()

    def pair_body(i, carry):
        for s in range(2):
            c = 2 * i + s
            in_copy(c, s).wait()

            @pl.when(c + 1 < _NCH)
            def _():
                in_copy(c + 1, 1 - s).start()

            @pl.when(c >= 2)
            def _():
                out_copy(c - 2, s).wait()

            @plsc.parallel_loop(0, _GR, 1, unroll=4)
            def g_body(g):
                r = g >> 4
                off = pl.multiple_of((g & 15) * _L, _L)
                ids = idx_v[s, r, pl.ds(off, _L)]
                rem = plsc.load_gather(lookup_v, [ids])
                rem = jnp.minimum(jnp.maximum(rem, 0), _ACTIVE - 1)
                rem10 = rem * _DIM
                for d in range(_DIM):
                    vals = plsc.load_gather(emb_v, [rem10 + d])
                    out_v[s, d, r, pl.ds(off, _L)] = vals

            out_copy(c, s).start()
        return carry

    lax.fori_loop(0, _NCH // 2, pair_body, 0, unroll=False)
    out_copy(_NCH - 2, 0).wait()
    out_copy(_NCH - 1, 1).wait()


def kernel(input_indices, lookup, emb_weight):
    idx_t = input_indices.T
    emb_flat = emb_weight.reshape(-1)
    mesh = plsc.VectorSubcoreMesh(core_axis_name="c", subcore_axis_name="s")
    f = pl.kernel(
        _sc_body,
        out_type=jax.ShapeDtypeStruct((_DIM, _HIST, _BATCH), jnp.float32),
        mesh=mesh,
        compiler_params=pltpu.CompilerParams(needs_layout_passes=False,
                                             use_tc_tiling_on_sc=True),
        scratch_types=[
            pltpu.VMEM((_VOCAB,), jnp.int32),
            pltpu.VMEM((_ACTIVE * _DIM,), jnp.float32),
            pltpu.VMEM((2, 8, _CW), jnp.int32),
            pltpu.VMEM((2, _DIM, 8, _CW), jnp.float32),
            pltpu.SemaphoreType.DMA,
            pltpu.SemaphoreType.DMA,
            pltpu.SemaphoreType.DMA,
            pltpu.SemaphoreType.DMA,
        ],
    )
    out = f(idx_t, lookup, emb_flat)
    return out.transpose(2, 1, 0)


# speedup vs baseline: 27.4874x; 1.0034x over previous
import jax
import jax.numpy as jnp
from jax import lax
from jax.experimental import pallas as pl
from jax.experimental.pallas import tpu as pltpu
from jax.experimental.pallas import tpu_sc as plsc

_VOCAB = 10000
_ACTIVE = 100
_DIM = 10
_L = 16
_NW = 32
_BATCH = 16384
_HIST = 200
_TW = _BATCH // _NW
_CW = 256
_NCH = (_HIST // 8) * (_TW // _CW)
_GR = 8 * (_CW // _L)


def _sc_body(idx_hbm, lookup_hbm, emb_hbm, out_hbm,
             lookup_v, emb_v, idx_v, out_v, sin0, sin1, sout0, sout1):
    wid = lax.axis_index("s") * 2 + lax.axis_index("c")
    b0 = wid * _TW
    sin = (sin0, sin1)
    sout = (sout0, sout1)

    pltpu.sync_copy(lookup_hbm, lookup_v)
    pltpu.sync_copy(emb_hbm, emb_v)

    def in_copy(c, s):
        th = c >> 1
        bc = b0 + (c & 1) * _CW
        return pltpu.make_async_copy(
            idx_hbm.at[pl.ds(th * 8, 8), pl.ds(bc, _CW)], idx_v.at[s], sin[s])

    def out_copy(c, s):
        th = c >> 1
        bc = b0 + (c & 1) * _CW
        return pltpu.make_async_copy(
            out_v.at[s], out_hbm.at[:, pl.ds(th * 8, 8), pl.ds(bc, _CW)],
            sout[s])

    in_copy(0, 0).start()

    def pair_body(i, carry):
        for s in range(2):
            c = 2 * i + s
            in_copy(c, s).wait()

            @pl.when(c + 1 < _NCH)
            def _():
                in_copy(c + 1, 1 - s).start()

            @pl.when(c >= 2)
            def _():
                out_copy(c - 2, s).wait()

            @plsc.parallel_loop(0, _GR, 1, unroll=8)
            def g_body(g):
                r = g >> 4
                off = pl.multiple_of((g & 15) * _L, _L)
                ids = idx_v[s, r, pl.ds(off, _L)]
                rem = plsc.load_gather(lookup_v, [ids])
                rem = jnp.minimum(jnp.maximum(rem, 0), _ACTIVE - 1)
                rem10 = rem * _DIM
                for d in range(_DIM):
                    vals = plsc.load_gather(emb_v, [rem10 + d])
                    out_v[s, d, r, pl.ds(off, _L)] = vals

            out_copy(c, s).start()
        return carry

    lax.fori_loop(0, _NCH // 2, pair_body, 0, unroll=False)
    out_copy(_NCH - 2, 0).wait()
    out_copy(_NCH - 1, 1).wait()


def kernel(input_indices, lookup, emb_weight):
    idx_t = input_indices.T
    emb_flat = emb_weight.reshape(-1)
    mesh = plsc.VectorSubcoreMesh(core_axis_name="c", subcore_axis_name="s")
    f = pl.kernel(
        _sc_body,
        out_type=jax.ShapeDtypeStruct((_DIM, _HIST, _BATCH), jnp.float32),
        mesh=mesh,
        compiler_params=pltpu.CompilerParams(needs_layout_passes=False,
                                             use_tc_tiling_on_sc=True),
        scratch_types=[
            pltpu.VMEM((_VOCAB,), jnp.int32),
            pltpu.VMEM((_ACTIVE * _DIM,), jnp.float32),
            pltpu.VMEM((2, 8, _CW), jnp.int32),
            pltpu.VMEM((2, _DIM, 8, _CW), jnp.float32),
            pltpu.SemaphoreType.DMA,
            pltpu.SemaphoreType.DMA,
            pltpu.SemaphoreType.DMA,
            pltpu.SemaphoreType.DMA,
        ],
    )
    out = f(idx_t, lookup, emb_flat)
    return out.transpose(2, 1, 0)
